# baseline (speedup 1.0000x reference)
import functools

import jax
import jax.numpy as jnp
from jax.experimental import pallas as pl
from jax.experimental.pallas import tpu as pltpu

_EPS = 1e-8


def _half(x, labels, ca_unit_b16, hi):
    raw = jnp.dot(
        x.astype(jnp.bfloat16), ca_unit_b16, preferred_element_type=jnp.float32
    )
    sumsq = jnp.sum(x * x, axis=1, keepdims=True)
    inv = jax.lax.rsqrt(jnp.maximum(sumsq, _EPS * _EPS))
    cos = raw * inv
    mt, k = cos.shape
    kidx = jax.lax.broadcasted_iota(jnp.int32, (mt, k), 1)
    uninit = jnp.where(labels == kidx, jnp.float32(1.0), jnp.float32(-1.0))
    return jnp.where(hi > 0.0, uninit, cos)


def _simrel_tile(ca_t_ref, xa_ref, xb_ref, lab_ref, out_ref, ca_unit_ref, hi_ref):
    @pl.when(pl.program_id(0) == 0)
    def _prep():
        ca_t = ca_t_ref[...]
        ca_sq = jnp.sum(ca_t * ca_t, axis=0, keepdims=True)
        ca_norm = jnp.sqrt(ca_sq)
        ca_unit = ca_t / jnp.maximum(ca_norm, _EPS)
        ca_unit_ref[...] = ca_unit.astype(jnp.bfloat16)
        has_inf = jnp.any(jnp.isinf(ca_t), axis=0, keepdims=True)
        hi_ref[...] = has_inf.astype(jnp.float32)

    ca_unit_b16 = ca_unit_ref[...]
    hi = hi_ref[...]
    mt = xa_ref.shape[0]
    lab = lab_ref[...]
    out_ref[:mt, :] = _half(xa_ref[...], lab[:mt, :], ca_unit_b16, hi)
    out_ref[mt:, :] = _half(xb_ref[...], lab[mt:, :], ca_unit_b16, hi)


@functools.partial(jax.jit, static_argnames=())
def kernel(inputs, labels, class_avgs):
    b, t, d = inputs.shape
    k = class_avgs.shape[0]
    m = b * t
    mt = 2048
    sup = 2 * mt
    n_tiles = m // sup

    x2 = inputs.reshape(m, d)
    lab2 = labels.astype(jnp.int32).reshape(m, 1)
    ca_t = class_avgs.T

    out = pl.pallas_call(
        _simrel_tile,
        grid=(n_tiles,),
        in_specs=[
            pl.BlockSpec((d, k), lambda i: (0, 0)),
            pl.BlockSpec((mt, d), lambda i: (2 * i, 0)),
            pl.BlockSpec((mt, d), lambda i: (2 * i + 1, 0)),
            pl.BlockSpec((sup, 1), lambda i: (i, 0)),
        ],
        out_specs=pl.BlockSpec((sup, k), lambda i: (i, 0)),
        out_shape=jax.ShapeDtypeStruct((m, k), jnp.float32),
        scratch_shapes=[
            pltpu.VMEM((d, k), jnp.bfloat16),
            pltpu.VMEM((1, k), jnp.float32),
        ],
        compiler_params=pltpu.CompilerParams(
            dimension_semantics=("arbitrary",),
        ),
    )(ca_t, x2, x2, lab2)
    return out.reshape(b, t, k)


# speedup vs baseline: 1.8293x; 1.8293x over previous
import functools

import jax
import jax.numpy as jnp
from jax.experimental import pallas as pl
from jax.experimental.pallas import tpu as pltpu

_EPS = 1e-8


def _half_t(x, ca_unit_t_b16, lab_row, hi):
    raw = jnp.dot(
        x.astype(jnp.bfloat16), ca_unit_t_b16, preferred_element_type=jnp.float32
    )
    raw_t = raw.T
    sumsq = jnp.sum(x * x, axis=1, keepdims=True)
    inv = jax.lax.rsqrt(jnp.maximum(sumsq, _EPS * _EPS)).reshape(1, -1)
    cos_t = raw_t * inv
    k, mt = cos_t.shape
    kidx = jax.lax.broadcasted_iota(jnp.int32, (k, mt), 0)
    uninit = jnp.where(lab_row == kidx, jnp.float32(1.0), jnp.float32(-1.0))
    return jnp.where(hi > 0.0, uninit, cos_t)


def _simrel_tile(ca_ref, xa_ref, xb_ref, lab_ref, out_ref, ca_unit_ref, hi_ref):
    nj = pl.num_programs(0) // lab_ref.shape[0]
    b = pl.program_id(0) // nj

    @pl.when(pl.program_id(0) == 0)
    def _prep():
        ca = ca_ref[...]
        ca_sq = jnp.sum(ca * ca, axis=1, keepdims=True)
        ca_norm = jnp.sqrt(ca_sq)
        ca_unit = ca / jnp.maximum(ca_norm, _EPS)
        ca_unit_ref[...] = ca_unit.T.astype(jnp.bfloat16)
        has_inf = jnp.any(jnp.isinf(ca), axis=1, keepdims=True)
        hi_ref[...] = has_inf.astype(jnp.float32)

    ca_unit_t_b16 = ca_unit_ref[...]
    hi = hi_ref[...]
    mt = xa_ref.shape[0]
    lab = lab_ref[pl.ds(b, 1), :]
    out_ref[0, :, :mt] = _half_t(xa_ref[...], ca_unit_t_b16, lab[:, :mt], hi)
    out_ref[0, :, mt:] = _half_t(xb_ref[...], ca_unit_t_b16, lab[:, mt:], hi)


@functools.partial(jax.jit, static_argnames=())
def kernel(inputs, labels, class_avgs):
    b, t, d = inputs.shape
    k = class_avgs.shape[0]
    m = b * t
    mt = 2048
    sup = 2 * mt
    nj = t // sup
    n_tiles = m // sup

    x2 = inputs.reshape(m, d)
    lab = labels.astype(jnp.int32)

    out_bkt = pl.pallas_call(
        _simrel_tile,
        grid=(n_tiles,),
        in_specs=[
            pl.BlockSpec((k, d), lambda i: (0, 0)),
            pl.BlockSpec((mt, d), lambda i: (2 * i, 0)),
            pl.BlockSpec((mt, d), lambda i: (2 * i + 1, 0)),
            pl.BlockSpec((b, sup), lambda i, nj=nj: (0, i % nj)),
        ],
        out_specs=pl.BlockSpec((1, k, sup), lambda i, nj=nj: (i // nj, 0, i % nj)),
        out_shape=jax.ShapeDtypeStruct((b, k, t), jnp.float32),
        scratch_shapes=[
            pltpu.VMEM((d, k), jnp.bfloat16),
            pltpu.VMEM((k, 1), jnp.float32),
        ],
        compiler_params=pltpu.CompilerParams(
            dimension_semantics=("arbitrary",),
        ),
    )(class_avgs, x2, x2, lab)
    return jnp.transpose(out_bkt, (0, 2, 1))
